# Initial kernel scaffold; baseline (speedup 1.0000x reference)
import jax
import jax.numpy as jnp
from jax.experimental import pallas as pl


def kernel(p, W0, b0, W1, b1, Wp, bp):
    raise NotImplementedError("write your pallas kernel here")


# speedup vs baseline: 1.0715x; 1.0715x over previous
import jax
import jax.numpy as jnp
from jax.experimental import pallas as pl

_B, _N, _G, _K = 8, 16384, 256, 32
_GT = 32


def _fps_indices(p, g):
    b, n, _ = p.shape
    dists0 = jnp.full((b, n), 1e10, dtype=p.dtype)
    far0 = jnp.zeros((b,), dtype=jnp.int32)

    def step(carry, _):
        dists, far = carry
        centroid = jnp.take_along_axis(p, far[:, None, None], axis=1)
        d = jnp.sum((p - centroid) ** 2, axis=-1)
        dists = jnp.minimum(dists, d)
        nxt = jnp.argmax(dists, axis=-1).astype(jnp.int32)
        return (dists, nxt), far

    (_, _), idxs = jax.lax.scan(step, (dists0, far0), None, length=g)
    return jnp.transpose(idxs)


def _mlp_block(npos8_ref, c8_ref, w0_ref, b0_ref, w1a_ref, w1b_ref, b1_ref,
               wp_ref, bp_ref, out_ref):
    npos = npos8_ref[...]
    c8 = c8_ref[...]
    rel = (npos.reshape(_GT, _K, 8) - c8[:, None, :]).reshape(_GT * _K, 8)
    nf = jnp.maximum(
        jnp.dot(npos, w0_ref[...], preferred_element_type=jnp.float32)
        + b0_ref[...], 0.0)
    h = jnp.maximum(
        jnp.dot(nf, w1a_ref[...], preferred_element_type=jnp.float32)
        + jnp.dot(rel, w1b_ref[...], preferred_element_type=jnp.float32)
        + b1_ref[...], 0.0)
    h = h.reshape(_GT, _K, 192)
    tok = jnp.concatenate([jnp.max(h, axis=1), jnp.mean(h, axis=1)], axis=-1)
    out_ref[...] = (jnp.dot(tok, wp_ref[...], preferred_element_type=jnp.float32)
                    + bp_ref[...])


def kernel(p, W0, b0, W1, b1, Wp, bp):
    center_idx = _fps_indices(jax.lax.stop_gradient(p), _G)
    center = jnp.take_along_axis(p, center_idx[:, :, None], axis=1)
    d2 = (jnp.sum(center ** 2, -1)[:, :, None]
          + jnp.sum(p ** 2, -1)[:, None, :]
          - 2.0 * jnp.einsum('bgc,bnc->bgn', center, p))
    neighbor_idx = jax.lax.top_k(-d2, _K)[1]
    npos = jax.vmap(lambda pts, idx: pts[idx])(p, neighbor_idx)

    npos8 = jnp.pad(npos.reshape(_B * _G * _K, 3), ((0, 0), (0, 5)))
    c8 = jnp.pad(center.reshape(_B * _G, 3), ((0, 0), (0, 5)))
    W1a = W1[:96]
    W1b = jnp.pad(W1[96:99], ((0, 5), (0, 0)))

    out = pl.pallas_call(
        _mlp_block,
        grid=(_B * _G // _GT,),
        in_specs=[
            pl.BlockSpec((_GT * _K, 8), lambda i: (i, 0)),
            pl.BlockSpec((_GT, 8), lambda i: (i, 0)),
            pl.BlockSpec((8, 96), lambda i: (0, 0)),
            pl.BlockSpec((1, 96), lambda i: (0, 0)),
            pl.BlockSpec((96, 192), lambda i: (0, 0)),
            pl.BlockSpec((8, 192), lambda i: (0, 0)),
            pl.BlockSpec((1, 192), lambda i: (0, 0)),
            pl.BlockSpec((384, 768), lambda i: (0, 0)),
            pl.BlockSpec((1, 768), lambda i: (0, 0)),
        ],
        out_specs=pl.BlockSpec((_GT, 768), lambda i: (i, 0)),
        out_shape=jax.ShapeDtypeStruct((_B * _G, 768), jnp.float32),
    )(npos8, c8,
      jnp.pad(W0, ((0, 5), (0, 0))), b0[None, :],
      W1a, W1b, b1[None, :], Wp, bp[None, :])

    return (out.reshape(_B, _G, 768), center)


# speedup vs baseline: 1.6192x; 1.5112x over previous
import jax
import jax.numpy as jnp
from jax.experimental import pallas as pl
from jax.experimental.pallas import tpu as pltpu

_B, _N, _G, _K = 8, 16384, 256, 32
_GT = 32
_NS = 128
_NL = 128


def _fps_kernel(px_ref, py_ref, pz_ref, cx_ref, cy_ref, cz_ref, dists_ref):
    idxf = (jax.lax.broadcasted_iota(jnp.int32, (1, _NS, _NL), 1) * _NL
            + jax.lax.broadcasted_iota(jnp.int32, (1, _NS, _NL), 2)
            ).astype(jnp.float32)
    px = px_ref[...]
    py = py_ref[...]
    pz = pz_ref[...]
    dists_ref[...] = jnp.full((_B, _NS, _NL), 1e10, jnp.float32)
    lane_g = jax.lax.broadcasted_iota(jnp.int32, (_B, _G), 1)

    def body(g, carry):
        cx, cy, cz, ax, ay, az = carry
        onehot = lane_g == g
        ax = jnp.where(onehot, cx.reshape(_B, 1), ax)
        ay = jnp.where(onehot, cy.reshape(_B, 1), ay)
        az = jnp.where(onehot, cz.reshape(_B, 1), az)
        dx = px - cx
        dy = py - cy
        dz = pz - cz
        d = dx * dx + dy * dy + dz * dz
        dn = jnp.minimum(dists_ref[...], d)
        dists_ref[...] = dn
        m = jnp.max(jnp.max(dn, axis=2, keepdims=True), axis=1, keepdims=True)
        eq = dn == m
        self_idx = jnp.where(eq, idxf, 3e7)
        sel = jnp.min(jnp.min(self_idx, axis=2, keepdims=True), axis=1,
                      keepdims=True)
        eq2 = idxf == sel
        zero = jnp.zeros((), jnp.float32)
        cxn = jnp.sum(jnp.sum(jnp.where(eq2, px, zero), axis=2, keepdims=True),
                      axis=1, keepdims=True)
        cyn = jnp.sum(jnp.sum(jnp.where(eq2, py, zero), axis=2, keepdims=True),
                      axis=1, keepdims=True)
        czn = jnp.sum(jnp.sum(jnp.where(eq2, pz, zero), axis=2, keepdims=True),
                      axis=1, keepdims=True)
        return (cxn, cyn, czn, ax, ay, az)

    cx0 = px_ref[:, 0:1, 0:1]
    cy0 = py_ref[:, 0:1, 0:1]
    cz0 = pz_ref[:, 0:1, 0:1]
    acc0 = jnp.zeros((_B, _G), jnp.float32)
    _, _, _, ax, ay, az = jax.lax.fori_loop(
        0, _G, body, (cx0, cy0, cz0, acc0, acc0, acc0))
    cx_ref[...] = ax
    cy_ref[...] = ay
    cz_ref[...] = az


def _fps_centers(p):
    px = p[:, :, 0].reshape(_B, _NS, _NL)
    py = p[:, :, 1].reshape(_B, _NS, _NL)
    pz = p[:, :, 2].reshape(_B, _NS, _NL)
    cx, cy, cz = pl.pallas_call(
        _fps_kernel,
        out_shape=[jax.ShapeDtypeStruct((_B, _G), jnp.float32)] * 3,
        scratch_shapes=[pltpu.VMEM((_B, _NS, _NL), jnp.float32)],
    )(px, py, pz)
    return jnp.stack([cx, cy, cz], axis=-1)


def _mlp_block(npos8_ref, c8_ref, w0_ref, b0_ref, w1a_ref, w1b_ref, b1_ref,
               wp_ref, bp_ref, out_ref):
    npos = npos8_ref[...]
    c8 = c8_ref[...]
    rel = (npos.reshape(_GT, _K, 8) - c8[:, None, :]).reshape(_GT * _K, 8)
    nf = jnp.maximum(
        jnp.dot(npos, w0_ref[...], preferred_element_type=jnp.float32)
        + b0_ref[...], 0.0)
    h = jnp.maximum(
        jnp.dot(nf, w1a_ref[...], preferred_element_type=jnp.float32)
        + jnp.dot(rel, w1b_ref[...], preferred_element_type=jnp.float32)
        + b1_ref[...], 0.0)
    h = h.reshape(_GT, _K, 192)
    tok = jnp.concatenate([jnp.max(h, axis=1), jnp.mean(h, axis=1)], axis=-1)
    out_ref[...] = (jnp.dot(tok, wp_ref[...], preferred_element_type=jnp.float32)
                    + bp_ref[...])


def kernel(p, W0, b0, W1, b1, Wp, bp):
    center = _fps_centers(p)
    d2 = (jnp.sum(center ** 2, -1)[:, :, None]
          + jnp.sum(p ** 2, -1)[:, None, :]
          - 2.0 * jnp.einsum('bgc,bnc->bgn', center, p))
    neighbor_idx = jax.lax.top_k(-d2, _K)[1]
    npos = jax.vmap(lambda pts, idx: pts[idx])(p, neighbor_idx)

    npos8 = jnp.pad(npos.reshape(_B * _G * _K, 3), ((0, 0), (0, 5)))
    c8 = jnp.pad(center.reshape(_B * _G, 3), ((0, 0), (0, 5)))
    W1a = W1[:96]
    W1b = jnp.pad(W1[96:99], ((0, 5), (0, 0)))

    out = pl.pallas_call(
        _mlp_block,
        grid=(_B * _G // _GT,),
        in_specs=[
            pl.BlockSpec((_GT * _K, 8), lambda i: (i, 0)),
            pl.BlockSpec((_GT, 8), lambda i: (i, 0)),
            pl.BlockSpec((8, 96), lambda i: (0, 0)),
            pl.BlockSpec((1, 96), lambda i: (0, 0)),
            pl.BlockSpec((96, 192), lambda i: (0, 0)),
            pl.BlockSpec((8, 192), lambda i: (0, 0)),
            pl.BlockSpec((1, 192), lambda i: (0, 0)),
            pl.BlockSpec((384, 768), lambda i: (0, 0)),
            pl.BlockSpec((1, 768), lambda i: (0, 0)),
        ],
        out_specs=pl.BlockSpec((_GT, 768), lambda i: (i, 0)),
        out_shape=jax.ShapeDtypeStruct((_B * _G, 768), jnp.float32),
    )(npos8, c8,
      jnp.pad(W0, ((0, 5), (0, 0))), b0[None, :],
      W1a, W1b, b1[None, :], Wp, bp[None, :])

    return (out.reshape(_B, _G, 768), center)


# speedup vs baseline: 1.6197x; 1.0003x over previous
import jax
import jax.numpy as jnp
from jax.experimental import pallas as pl
from jax.experimental.pallas import tpu as pltpu

_B, _N, _G, _K = 8, 16384, 256, 32
_GT = 32
_NS = 128
_NL = 128


def _fps_kernel(px_ref, py_ref, pz_ref, cx_ref, cy_ref, cz_ref, dists_ref):
    idxf = (jax.lax.broadcasted_iota(jnp.int32, (1, _NS, _NL), 1) * _NL
            + jax.lax.broadcasted_iota(jnp.int32, (1, _NS, _NL), 2)
            ).astype(jnp.float32)
    px = px_ref[...]
    py = py_ref[...]
    pz = pz_ref[...]
    dists_ref[...] = jnp.full((_B, _NS, _NL), 1e10, jnp.float32)
    lane_g = jax.lax.broadcasted_iota(jnp.int32, (_B, _G), 1)

    def body(g, carry):
        cx, cy, cz, ax, ay, az = carry
        onehot = lane_g == g
        ax = jnp.where(onehot, cx.reshape(_B, 1), ax)
        ay = jnp.where(onehot, cy.reshape(_B, 1), ay)
        az = jnp.where(onehot, cz.reshape(_B, 1), az)
        dx = px - cx
        dy = py - cy
        dz = pz - cz
        d = dx * dx + dy * dy + dz * dz
        dn = jnp.minimum(dists_ref[...], d)
        dists_ref[...] = dn
        m = jnp.max(jnp.max(dn, axis=2, keepdims=True), axis=1, keepdims=True)
        eq = dn == m
        sel = jnp.min(jnp.min(jnp.where(eq, idxf, 3e7), axis=2, keepdims=True),
                      axis=1, keepdims=True)
        eq2 = idxf == sel
        zero = jnp.zeros((), jnp.float32)
        cxn = jnp.sum(jnp.sum(jnp.where(eq2, px, zero), axis=2, keepdims=True),
                      axis=1, keepdims=True)
        cyn = jnp.sum(jnp.sum(jnp.where(eq2, py, zero), axis=2, keepdims=True),
                      axis=1, keepdims=True)
        czn = jnp.sum(jnp.sum(jnp.where(eq2, pz, zero), axis=2, keepdims=True),
                      axis=1, keepdims=True)
        return (cxn, cyn, czn, ax, ay, az)

    cx0 = px_ref[:, 0:1, 0:1]
    cy0 = py_ref[:, 0:1, 0:1]
    cz0 = pz_ref[:, 0:1, 0:1]
    acc0 = jnp.zeros((_B, _G), jnp.float32)
    _, _, _, ax, ay, az = jax.lax.fori_loop(
        0, _G, body, (cx0, cy0, cz0, acc0, acc0, acc0))
    cx_ref[...] = ax
    cy_ref[...] = ay
    cz_ref[...] = az


def _fps_centers(p):
    px = p[:, :, 0].reshape(_B, _NS, _NL)
    py = p[:, :, 1].reshape(_B, _NS, _NL)
    pz = p[:, :, 2].reshape(_B, _NS, _NL)
    cx, cy, cz = pl.pallas_call(
        _fps_kernel,
        out_shape=[jax.ShapeDtypeStruct((_B, _G), jnp.float32)] * 3,
        scratch_shapes=[pltpu.VMEM((_B, _NS, _NL), jnp.float32)],
    )(px, py, pz)
    return cx, cy, cz


def _mlp_block(npos8_ref, c8_ref, w0_ref, b0_ref, w1a_ref, w1b_ref, b1_ref,
               wp_ref, bp_ref, out_ref):
    npos = npos8_ref[...]
    c8 = c8_ref[...]
    rel = (npos.reshape(_GT, _K, 8) - c8[:, None, :]).reshape(_GT * _K, 8)
    nf = jnp.maximum(
        jnp.dot(npos, w0_ref[...], preferred_element_type=jnp.float32)
        + b0_ref[...], 0.0)
    h = jnp.maximum(
        jnp.dot(nf, w1a_ref[...], preferred_element_type=jnp.float32)
        + jnp.dot(rel, w1b_ref[...], preferred_element_type=jnp.float32)
        + b1_ref[...], 0.0)
    h = h.reshape(_GT, _K, 192)
    tok = jnp.concatenate([jnp.max(h, axis=1), jnp.mean(h, axis=1)], axis=-1)
    out_ref[...] = (jnp.dot(tok, wp_ref[...], preferred_element_type=jnp.float32)
                    + bp_ref[...])


def kernel(p, W0, b0, W1, b1, Wp, bp):
    cx, cy, cz = _fps_centers(p)
    center = jnp.stack([cx, cy, cz], axis=-1)

    d2 = (jnp.sum(center ** 2, -1)[:, :, None]
          + jnp.sum(p ** 2, -1)[:, None, :]
          - 2.0 * jnp.einsum('bgc,bnc->bgn', center, p))
    neighbor_idx = jax.lax.top_k(-d2, _K)[1]
    npos = jax.vmap(lambda pts, idx: pts[idx])(p, neighbor_idx)

    npos8 = jnp.pad(npos.reshape(_B * _G * _K, 3), ((0, 0), (0, 5)))
    c8 = jnp.pad(center.reshape(_B * _G, 3), ((0, 0), (0, 5)))
    W1a = W1[:96]
    W1b = jnp.pad(W1[96:99], ((0, 5), (0, 0)))

    out = pl.pallas_call(
        _mlp_block,
        grid=(_B * _G // _GT,),
        in_specs=[
            pl.BlockSpec((_GT * _K, 8), lambda i: (i, 0)),
            pl.BlockSpec((_GT, 8), lambda i: (i, 0)),
            pl.BlockSpec((8, 96), lambda i: (0, 0)),
            pl.BlockSpec((1, 96), lambda i: (0, 0)),
            pl.BlockSpec((96, 192), lambda i: (0, 0)),
            pl.BlockSpec((8, 192), lambda i: (0, 0)),
            pl.BlockSpec((1, 192), lambda i: (0, 0)),
            pl.BlockSpec((384, 768), lambda i: (0, 0)),
            pl.BlockSpec((1, 768), lambda i: (0, 0)),
        ],
        out_specs=pl.BlockSpec((_GT, 768), lambda i: (i, 0)),
        out_shape=jax.ShapeDtypeStruct((_B * _G, 768), jnp.float32),
    )(npos8, c8,
      jnp.pad(W0, ((0, 5), (0, 0))), b0[None, :],
      W1a, W1b, b1[None, :], Wp, bp[None, :])

    return (out.reshape(_B, _G, 768), center)
